# baseline (speedup 1.0000x reference)
import dataclasses
import functools

import jax
import jax.numpy as jnp
from jax import lax
from jax.experimental import pallas as pl
from jax.experimental.pallas import tpu as pltpu
from jax.experimental.pallas import tpu_sc as plsc

_B = 4096
_D = 64
_GAMMA = 12.0
_NC = 2
_NS = 16
_NW = _NC * _NS
_BPW = _B // _NW
_L = 16


def _rsqrt(x):
    i = plsc.bitcast(x, jnp.int32)
    i = jnp.int32(0x5F3759DF) - (i >> 1)
    y = plsc.bitcast(i, jnp.float32)
    for _ in range(3):
        y = y * (1.5 - 0.5 * x * y * y)
    return y


def _sc_body(smp_hbm, ent0_hbm, ent1_hbm, ent4_hbm, ent5_hbm,
             rel0_hbm, rel1_hbm, rel3_hbm, rel4_hbm, w_hbm,
             out_hbm,
             smp_v, hi_v, ri_v, ti_v, w_v,
             h0_v, t0_v, h1_v, t1_v, h4_v, t4_v, h5_v, t5_v,
             r0_v, r1_v, r3_v, r4_v,
             s1_v, s2_v, s3_v, s4_v, out_v, sem):
    wid = lax.axis_index("s") * _NC + lax.axis_index("c")
    base = wid * _BPW

    pltpu.sync_copy(smp_hbm.at[pl.ds(base, _BPW)], smp_v)
    pltpu.sync_copy(w_hbm, w_v)

    for j in range(_BPW // _L):
        rows = lax.iota(jnp.int32, _L) + jnp.int32(j * _L)
        for col, dst in ((0, hi_v), (1, ri_v), (2, ti_v)):
            cols = jnp.full((_L,), col, jnp.int32)
            dst[pl.ds(j * _L, _L)] = plsc.load_gather(smp_v, [rows, cols])

    gathers = (
        (ent0_hbm, hi_v, h0_v), (ent0_hbm, ti_v, t0_v),
        (ent1_hbm, hi_v, h1_v), (ent1_hbm, ti_v, t1_v),
        (ent4_hbm, hi_v, h4_v), (ent4_hbm, ti_v, t4_v),
        (ent5_hbm, hi_v, h5_v), (ent5_hbm, ti_v, t5_v),
        (rel0_hbm, ri_v, r0_v), (rel1_hbm, ri_v, r1_v),
        (rel3_hbm, ri_v, r3_v), (rel4_hbm, ri_v, r4_v),
    )
    copies = [pltpu.async_copy(tbl.at[idx], dst, sem)
              for tbl, idx, dst in gathers]
    for c in copies:
        c.wait()

    last = lax.iota(jnp.int32, _L) == (_L - 1)

    def row(i, carry):
        a1 = a2 = a3 = a4 = jnp.zeros((_L,), jnp.float32)
        for c in range(_D // _L):
            sl = pl.ds(c * _L, _L)
            d1 = h0_v[i, sl] + r0_v[i, sl] - t0_v[i, sl]
            d2 = t1_v[i, sl] + r1_v[i, sl] - h1_v[i, sl]
            d3 = h4_v[i, sl] + t4_v[i, sl] - r3_v[i, sl]
            d4 = h5_v[i, sl] * r4_v[i, sl] - t5_v[i, sl]
            a1 = a1 + d1 * d1
            a2 = a2 + d2 * d2
            a3 = a3 + d3 * d3
            a4 = a4 + d4 * d4
        iv = jnp.full((_L,), 0, jnp.int32) + i
        plsc.store_scatter(s1_v, [iv], jnp.cumsum(a1), mask=last)
        plsc.store_scatter(s2_v, [iv], jnp.cumsum(a2), mask=last)
        plsc.store_scatter(s3_v, [iv], jnp.cumsum(a3), mask=last)
        plsc.store_scatter(s4_v, [iv], jnp.cumsum(a4), mask=last)
        return carry

    lax.fori_loop(0, _BPW, row, 0)

    wv = w_v[pl.ds(0, _L)]
    w0, w1, w2, w3 = wv[0], wv[1], wv[2], wv[3]
    for j in range(_BPW // _L):
        sl = pl.ds(j * _L, _L)
        s1, s2, s3, s4 = s1_v[sl], s2_v[sl], s3_v[sl], s4_v[sl]
        n1 = s1 * _rsqrt(s1)
        n2 = s2 * _rsqrt(s2)
        n3 = s3 * _rsqrt(s3)
        n4 = s4 * _rsqrt(s4)
        out_v[sl] = _GAMMA - (w0 * n1 + w1 * n2 + w2 * n3 + w3 * n4)

    pltpu.sync_copy(out_v, out_hbm.at[pl.ds(base, _BPW)])


@jax.jit
def kernel(sample, ent0, ent1, ent4, ent5, rel0, rel1, rel3, rel4, w):
    cp = pltpu.CompilerParams(use_tc_tiling_on_sc=False)
    if "needs_layout_passes" in pltpu.CompilerParams.__dataclass_fields__:
        cp = dataclasses.replace(cp, needs_layout_passes=False)
    run = pl.kernel(
        _sc_body,
        out_type=jax.ShapeDtypeStruct((_B,), jnp.float32),
        mesh=plsc.VectorSubcoreMesh(core_axis_name="c", subcore_axis_name="s"),
        compiler_params=cp,
        scratch_types=[
            pltpu.VMEM((_BPW, 3), jnp.int32),
            pltpu.VMEM((_BPW,), jnp.int32),
            pltpu.VMEM((_BPW,), jnp.int32),
            pltpu.VMEM((_BPW,), jnp.int32),
            pltpu.VMEM((16,), jnp.float32),
        ] + [pltpu.VMEM((_BPW, _D), jnp.float32) for _ in range(12)] + [
            pltpu.VMEM((_BPW,), jnp.float32),
            pltpu.VMEM((_BPW,), jnp.float32),
            pltpu.VMEM((_BPW,), jnp.float32),
            pltpu.VMEM((_BPW,), jnp.float32),
            pltpu.VMEM((_BPW,), jnp.float32),
            pltpu.SemaphoreType.DMA,
        ],
    )
    w16 = jnp.pad(w, (0, 16 - w.shape[0]))
    return run(sample, ent0, ent1, ent4, ent5, rel0, rel1, rel3, rel4, w16)


# speedup vs baseline: 1.2718x; 1.2718x over previous
import dataclasses
import functools

import jax
import jax.numpy as jnp
from jax import lax
from jax.experimental import pallas as pl
from jax.experimental.pallas import tpu as pltpu
from jax.experimental.pallas import tpu_sc as plsc

_B = 4096
_D = 64
_GAMMA = 12.0
_NC = 2
_NS = 16
_NW = _NC * _NS
_BPW = _B // _NW
_L = 16
_NG = _BPW // _L
_GROUP_ROWS = 6 * _L


def _rsqrt(x):
    i = plsc.bitcast(x, jnp.int32)
    i = jnp.int32(0x5F3759DF) - (i >> 1)
    y = plsc.bitcast(i, jnp.float32)
    for _ in range(3):
        y = y * (1.5 - 0.5 * x * y * y)
    return y


def _sc_body(h_hbm, r_hbm, t_hbm, ent0_hbm, ent1_hbm, ent4_hbm, ent5_hbm,
             rel0_hbm, rel1_hbm, rel3_hbm, rel4_hbm, w_hbm,
             out_hbm,
             hi_v, ri_v, ti_v, w_v,
             b0, b1, b2, b3, b4, b5,
             s1_v, s2_v, s3_v, s4_v, out_v,
             sem_ent):
    cid = lax.axis_index("c")
    sid = lax.axis_index("s")
    wid = sid * _NC + cid
    base = wid * _BPW

    pltpu.sync_copy(h_hbm.at[pl.ds(base, _BPW)], hi_v)
    pltpu.sync_copy(r_hbm.at[pl.ds(base, _BPW)], ri_v)
    pltpu.sync_copy(t_hbm.at[pl.ds(base, _BPW)], ti_v)
    pltpu.sync_copy(w_hbm, w_v)

    last = lax.iota(jnp.int32, _L) == (_L - 1)

    def _drain_group():
        pltpu.make_async_copy(ent0_hbm.at[pl.ds(0, _GROUP_ROWS), :],
                              b0.at[pl.ds(0, _GROUP_ROWS), :], sem_ent).wait()

    def _run_pass(tables, compute_row):
        def fire_group(g, carry):
            hv = hi_v[pl.ds(g * _L, _L)]
            rv = ri_v[pl.ds(g * _L, _L)]
            tv = ti_v[pl.ds(g * _L, _L)]
            for k in range(_L):
                scalars = (hv[k], rv[k], tv[k])
                row = g * _L + k
                for tbl, kind, buf in tables:
                    pltpu.async_copy(tbl.at[pl.ds(scalars[kind], 1), :],
                                     buf.at[pl.ds(row, 1), :], sem_ent)
            return carry

        fire_group(0, 0)

        def fire_and_drain(g, carry):
            fire_group(g, 0)
            _drain_group()
            return carry

        lax.fori_loop(1, _NG, fire_and_drain, 0)
        _drain_group()
        lax.fori_loop(0, _BPW, compute_row, 0)

    def row_a(i, carry):
        a1 = a2 = jnp.zeros((_L,), jnp.float32)
        for c in range(_D // _L):
            sl = pl.ds(c * _L, _L)
            d1 = b0[i, sl] + b4[i, sl] - b1[i, sl]
            d2 = b3[i, sl] + b5[i, sl] - b2[i, sl]
            a1 = a1 + d1 * d1
            a2 = a2 + d2 * d2
        iv = jnp.full((_L,), 0, jnp.int32) + i
        plsc.store_scatter(s1_v, [iv], jnp.cumsum(a1), mask=last)
        plsc.store_scatter(s2_v, [iv], jnp.cumsum(a2), mask=last)
        return carry

    _run_pass(((ent0_hbm, 0, b0), (ent0_hbm, 2, b1),
               (ent1_hbm, 0, b2), (ent1_hbm, 2, b3),
               (rel0_hbm, 1, b4), (rel1_hbm, 1, b5)), row_a)

    def row_b(i, carry):
        a3 = a4 = jnp.zeros((_L,), jnp.float32)
        for c in range(_D // _L):
            sl = pl.ds(c * _L, _L)
            d3 = b0[i, sl] + b1[i, sl] - b4[i, sl]
            d4 = b2[i, sl] * b5[i, sl] - b3[i, sl]
            a3 = a3 + d3 * d3
            a4 = a4 + d4 * d4
        iv = jnp.full((_L,), 0, jnp.int32) + i
        plsc.store_scatter(s3_v, [iv], jnp.cumsum(a3), mask=last)
        plsc.store_scatter(s4_v, [iv], jnp.cumsum(a4), mask=last)
        return carry

    _run_pass(((ent4_hbm, 0, b0), (ent4_hbm, 2, b1),
               (ent5_hbm, 0, b2), (ent5_hbm, 2, b3),
               (rel3_hbm, 1, b4), (rel4_hbm, 1, b5)), row_b)

    wv = w_v[pl.ds(0, _L)]
    w0, w1, w2, w3 = wv[0], wv[1], wv[2], wv[3]
    for j in range(_NG):
        sl = pl.ds(j * _L, _L)
        s1, s2, s3, s4 = s1_v[sl], s2_v[sl], s3_v[sl], s4_v[sl]
        n1 = s1 * _rsqrt(s1)
        n2 = s2 * _rsqrt(s2)
        n3 = s3 * _rsqrt(s3)
        n4 = s4 * _rsqrt(s4)
        out_v[sl] = _GAMMA - (w0 * n1 + w1 * n2 + w2 * n3 + w3 * n4)

    pltpu.sync_copy(out_v, out_hbm.at[pl.ds(base, _BPW)])


@jax.jit
def kernel(sample, ent0, ent1, ent4, ent5, rel0, rel1, rel3, rel4, w):
    cp = pltpu.CompilerParams()
    if "needs_layout_passes" in pltpu.CompilerParams.__dataclass_fields__:
        cp = dataclasses.replace(cp, needs_layout_passes=False)
    run = pl.kernel(
        _sc_body,
        out_type=jax.ShapeDtypeStruct((_B,), jnp.float32),
        mesh=plsc.VectorSubcoreMesh(core_axis_name="c", subcore_axis_name="s"),
        compiler_params=cp,
        scratch_types=[
            pltpu.VMEM((_BPW,), jnp.int32),
            pltpu.VMEM((_BPW,), jnp.int32),
            pltpu.VMEM((_BPW,), jnp.int32),
            pltpu.VMEM((16,), jnp.float32),
        ] + [pltpu.VMEM((_BPW, _D), jnp.float32) for _ in range(6)] + [
            pltpu.VMEM((_BPW,), jnp.float32),
            pltpu.VMEM((_BPW,), jnp.float32),
            pltpu.VMEM((_BPW,), jnp.float32),
            pltpu.VMEM((_BPW,), jnp.float32),
            pltpu.VMEM((_BPW,), jnp.float32),
            pltpu.SemaphoreType.DMA,
        ],
    )
    w16 = jnp.pad(w, (0, 16 - w.shape[0]))
    h_i = sample[:, 0]
    r_i = sample[:, 1]
    t_i = sample[:, 2]
    return run(h_i, r_i, t_i, ent0, ent1, ent4, ent5, rel0, rel1, rel3, rel4,
               w16)


# speedup vs baseline: 1.2980x; 1.0206x over previous
import dataclasses
import functools

import jax
import jax.numpy as jnp
from jax import lax
from jax.experimental import pallas as pl
from jax.experimental.pallas import tpu as pltpu
from jax.experimental.pallas import tpu_sc as plsc

_B = 4096
_D = 64
_GAMMA = 12.0
_NC = 2
_NS = 16
_NW = _NC * _NS
_BPW = _B // _NW
_L = 16
_NG = _BPW // _L
_GR = 6 * _L


def _rsqrt(x):
    i = plsc.bitcast(x, jnp.int32)
    i = jnp.int32(0x5F3759DF) - (i >> 1)
    y = plsc.bitcast(i, jnp.float32)
    for _ in range(3):
        y = y * (1.5 - 0.5 * x * y * y)
    return y


def _sc_body(h_hbm, r_hbm, t_hbm, ent0_hbm, ent1_hbm, ent4_hbm, ent5_hbm,
             rel0_hbm, rel1_hbm, rel3_hbm, rel4_hbm, w_hbm,
             out_hbm,
             hi_v, ri_v, ti_v, w_v,
             b0, b1, b2, b3, b4, b5,
             s1_v, s2_v, s3_v, s4_v, out_v,
             sem_ent):
    cid = lax.axis_index("c")
    sid = lax.axis_index("s")
    wid = sid * _NC + cid
    base = wid * _BPW

    pltpu.sync_copy(h_hbm.at[pl.ds(base, _BPW)], hi_v)
    pltpu.sync_copy(r_hbm.at[pl.ds(base, _BPW)], ri_v)
    pltpu.sync_copy(t_hbm.at[pl.ds(base, _BPW)], ti_v)
    pltpu.sync_copy(w_hbm, w_v)

    last = lax.iota(jnp.int32, _L) == (_L - 1)

    def _drain_group():
        pltpu.make_async_copy(ent0_hbm.at[pl.ds(0, _GR), :],
                              b0.at[pl.ds(0, _GR), :], sem_ent).wait()

    def _run_pass(tables, compute_group):
        def fire_group(g):
            hv = hi_v[pl.ds(g * _L, _L)]
            rv = ri_v[pl.ds(g * _L, _L)]
            tv = ti_v[pl.ds(g * _L, _L)]
            for k in range(_L):
                scalars = (hv[k], rv[k], tv[k])
                row = g * _L + k
                for tbl, kind, buf in tables:
                    pltpu.async_copy(tbl.at[pl.ds(scalars[kind], 1), :],
                                     buf.at[pl.ds(row, 1), :], sem_ent)

        fire_group(0)

        def step(g, carry):
            fire_group(g)
            _drain_group()
            compute_group(g - 1)
            return carry

        lax.fori_loop(1, _NG, step, 0)
        _drain_group()
        compute_group(_NG - 1)

    def rows_a(gg):
        def row(i, carry):
            a1 = a2 = jnp.zeros((_L,), jnp.float32)
            for c in range(_D // _L):
                sl = pl.ds(c * _L, _L)
                d1 = b0[i, sl] + b4[i, sl] - b1[i, sl]
                d2 = b3[i, sl] + b5[i, sl] - b2[i, sl]
                a1 = a1 + d1 * d1
                a2 = a2 + d2 * d2
            iv = jnp.full((_L,), 0, jnp.int32) + i
            plsc.store_scatter(s1_v, [iv], jnp.cumsum(a1), mask=last)
            plsc.store_scatter(s2_v, [iv], jnp.cumsum(a2), mask=last)
            return carry

        lax.fori_loop(gg * _L, gg * _L + _L, row, 0)

    _run_pass(((ent0_hbm, 0, b0), (ent0_hbm, 2, b1),
               (ent1_hbm, 0, b2), (ent1_hbm, 2, b3),
               (rel0_hbm, 1, b4), (rel1_hbm, 1, b5)), rows_a)

    def rows_b(gg):
        def row(i, carry):
            a3 = a4 = jnp.zeros((_L,), jnp.float32)
            for c in range(_D // _L):
                sl = pl.ds(c * _L, _L)
                d3 = b0[i, sl] + b1[i, sl] - b4[i, sl]
                d4 = b2[i, sl] * b5[i, sl] - b3[i, sl]
                a3 = a3 + d3 * d3
                a4 = a4 + d4 * d4
            iv = jnp.full((_L,), 0, jnp.int32) + i
            plsc.store_scatter(s3_v, [iv], jnp.cumsum(a3), mask=last)
            plsc.store_scatter(s4_v, [iv], jnp.cumsum(a4), mask=last)
            return carry

        lax.fori_loop(gg * _L, gg * _L + _L, row, 0)

    _run_pass(((ent4_hbm, 0, b0), (ent4_hbm, 2, b1),
               (ent5_hbm, 0, b2), (ent5_hbm, 2, b3),
               (rel3_hbm, 1, b4), (rel4_hbm, 1, b5)), rows_b)

    wv = w_v[pl.ds(0, _L)]
    w0, w1, w2, w3 = wv[0], wv[1], wv[2], wv[3]
    for j in range(_NG):
        sl = pl.ds(j * _L, _L)
        s1, s2, s3, s4 = s1_v[sl], s2_v[sl], s3_v[sl], s4_v[sl]
        n1 = s1 * _rsqrt(s1)
        n2 = s2 * _rsqrt(s2)
        n3 = s3 * _rsqrt(s3)
        n4 = s4 * _rsqrt(s4)
        out_v[sl] = _GAMMA - (w0 * n1 + w1 * n2 + w2 * n3 + w3 * n4)

    pltpu.sync_copy(out_v, out_hbm.at[pl.ds(base, _BPW)])


@jax.jit
def kernel(sample, ent0, ent1, ent4, ent5, rel0, rel1, rel3, rel4, w):
    cp = pltpu.CompilerParams()
    if "needs_layout_passes" in pltpu.CompilerParams.__dataclass_fields__:
        cp = dataclasses.replace(cp, needs_layout_passes=False)
    run = pl.kernel(
        _sc_body,
        out_type=jax.ShapeDtypeStruct((_B,), jnp.float32),
        mesh=plsc.VectorSubcoreMesh(core_axis_name="c", subcore_axis_name="s"),
        compiler_params=cp,
        scratch_types=[
            pltpu.VMEM((_BPW,), jnp.int32),
            pltpu.VMEM((_BPW,), jnp.int32),
            pltpu.VMEM((_BPW,), jnp.int32),
            pltpu.VMEM((16,), jnp.float32),
        ] + [pltpu.VMEM((_BPW, _D), jnp.float32) for _ in range(6)] + [
            pltpu.VMEM((_BPW,), jnp.float32),
            pltpu.VMEM((_BPW,), jnp.float32),
            pltpu.VMEM((_BPW,), jnp.float32),
            pltpu.VMEM((_BPW,), jnp.float32),
            pltpu.VMEM((_BPW,), jnp.float32),
            pltpu.SemaphoreType.DMA,
        ],
    )
    w16 = jnp.pad(w, (0, 16 - w.shape[0]))
    h_i = sample[:, 0]
    r_i = sample[:, 1]
    t_i = sample[:, 2]
    return run(h_i, r_i, t_i, ent0, ent1, ent4, ent5, rel0, rel1, rel3, rel4,
               w16)


# speedup vs baseline: 1.3012x; 1.0025x over previous
import dataclasses
import functools

import jax
import jax.numpy as jnp
from jax import lax
from jax.experimental import pallas as pl
from jax.experimental.pallas import tpu as pltpu
from jax.experimental.pallas import tpu_sc as plsc

_B = 4096
_D = 64
_GAMMA = 12.0
_NC = 2
_NS = 16
_NW = _NC * _NS
_BPW = _B // _NW
_L = 16
_NG = _BPW // _L
_GR = 6 * _L


def _rsqrt(x):
    i = plsc.bitcast(x, jnp.int32)
    i = jnp.int32(0x5F3759DF) - (i >> 1)
    y = plsc.bitcast(i, jnp.float32)
    for _ in range(3):
        y = y * (1.5 - 0.5 * x * y * y)
    return y


def _sc_body(h_hbm, r_hbm, t_hbm, ent0_hbm, ent1_hbm, ent4_hbm, ent5_hbm,
             rel0_hbm, rel1_hbm, rel3_hbm, rel4_hbm, w_hbm,
             out_hbm,
             hi_v, ri_v, ti_v, w_v,
             b0, b1, b2, b3, b4, b5,
             s1_v, s2_v, s3_v, s4_v, out_v,
             sem_ent):
    cid = lax.axis_index("c")
    sid = lax.axis_index("s")
    wid = sid * _NC + cid
    base = wid * _BPW

    pltpu.sync_copy(h_hbm.at[pl.ds(base, _BPW)], hi_v)
    pltpu.sync_copy(r_hbm.at[pl.ds(base, _BPW)], ri_v)
    pltpu.sync_copy(t_hbm.at[pl.ds(base, _BPW)], ti_v)
    pltpu.sync_copy(w_hbm, w_v)

    last = lax.iota(jnp.int32, _L) == (_L - 1)

    def _drain_group():
        pltpu.make_async_copy(ent0_hbm.at[pl.ds(0, _GR), :],
                              b0.at[pl.ds(0, _GR), :], sem_ent).wait()

    def _run_pass(tables, compute_group):
        def fire_group(g):
            hv = hi_v[pl.ds(g * _L, _L)]
            rv = ri_v[pl.ds(g * _L, _L)]
            tv = ti_v[pl.ds(g * _L, _L)]
            for k in range(_L):
                scalars = (hv[k], rv[k], tv[k])
                row = g * _L + k
                for tbl, kind, buf in tables:
                    pltpu.async_copy(tbl.at[pl.ds(scalars[kind], 1), :],
                                     buf.at[pl.ds(row, 1), :], sem_ent)

        fire_group(0)
        fire_group(1)

        def step(g, carry):
            fire_group(g)
            _drain_group()
            compute_group(g - 2)
            return carry

        lax.fori_loop(2, _NG, step, 0)
        _drain_group()
        compute_group(_NG - 2)
        _drain_group()
        compute_group(_NG - 1)

    def rows_a(gg):
        def row(i, carry):
            a1 = a2 = jnp.zeros((_L,), jnp.float32)
            for c in range(_D // _L):
                sl = pl.ds(c * _L, _L)
                d1 = b0[i, sl] + b4[i, sl] - b1[i, sl]
                d2 = b3[i, sl] + b5[i, sl] - b2[i, sl]
                a1 = a1 + d1 * d1
                a2 = a2 + d2 * d2
            iv = jnp.full((_L,), 0, jnp.int32) + i
            plsc.store_scatter(s1_v, [iv], jnp.cumsum(a1), mask=last)
            plsc.store_scatter(s2_v, [iv], jnp.cumsum(a2), mask=last)
            return carry

        lax.fori_loop(gg * _L, gg * _L + _L, row, 0)

    _run_pass(((ent0_hbm, 0, b0), (ent0_hbm, 2, b1),
               (ent1_hbm, 0, b2), (ent1_hbm, 2, b3),
               (rel0_hbm, 1, b4), (rel1_hbm, 1, b5)), rows_a)

    def rows_b(gg):
        def row(i, carry):
            a3 = a4 = jnp.zeros((_L,), jnp.float32)
            for c in range(_D // _L):
                sl = pl.ds(c * _L, _L)
                d3 = b0[i, sl] + b1[i, sl] - b4[i, sl]
                d4 = b2[i, sl] * b5[i, sl] - b3[i, sl]
                a3 = a3 + d3 * d3
                a4 = a4 + d4 * d4
            iv = jnp.full((_L,), 0, jnp.int32) + i
            plsc.store_scatter(s3_v, [iv], jnp.cumsum(a3), mask=last)
            plsc.store_scatter(s4_v, [iv], jnp.cumsum(a4), mask=last)
            return carry

        lax.fori_loop(gg * _L, gg * _L + _L, row, 0)

    _run_pass(((ent4_hbm, 0, b0), (ent4_hbm, 2, b1),
               (ent5_hbm, 0, b2), (ent5_hbm, 2, b3),
               (rel3_hbm, 1, b4), (rel4_hbm, 1, b5)), rows_b)

    wv = w_v[pl.ds(0, _L)]
    w0, w1, w2, w3 = wv[0], wv[1], wv[2], wv[3]
    for j in range(_NG):
        sl = pl.ds(j * _L, _L)
        s1, s2, s3, s4 = s1_v[sl], s2_v[sl], s3_v[sl], s4_v[sl]
        n1 = s1 * _rsqrt(s1)
        n2 = s2 * _rsqrt(s2)
        n3 = s3 * _rsqrt(s3)
        n4 = s4 * _rsqrt(s4)
        out_v[sl] = _GAMMA - (w0 * n1 + w1 * n2 + w2 * n3 + w3 * n4)

    pltpu.sync_copy(out_v, out_hbm.at[pl.ds(base, _BPW)])


@jax.jit
def kernel(sample, ent0, ent1, ent4, ent5, rel0, rel1, rel3, rel4, w):
    cp = pltpu.CompilerParams()
    if "needs_layout_passes" in pltpu.CompilerParams.__dataclass_fields__:
        cp = dataclasses.replace(cp, needs_layout_passes=False)
    run = pl.kernel(
        _sc_body,
        out_type=jax.ShapeDtypeStruct((_B,), jnp.float32),
        mesh=plsc.VectorSubcoreMesh(core_axis_name="c", subcore_axis_name="s"),
        compiler_params=cp,
        scratch_types=[
            pltpu.VMEM((_BPW,), jnp.int32),
            pltpu.VMEM((_BPW,), jnp.int32),
            pltpu.VMEM((_BPW,), jnp.int32),
            pltpu.VMEM((16,), jnp.float32),
        ] + [pltpu.VMEM((_BPW, _D), jnp.float32) for _ in range(6)] + [
            pltpu.VMEM((_BPW,), jnp.float32),
            pltpu.VMEM((_BPW,), jnp.float32),
            pltpu.VMEM((_BPW,), jnp.float32),
            pltpu.VMEM((_BPW,), jnp.float32),
            pltpu.VMEM((_BPW,), jnp.float32),
            pltpu.SemaphoreType.DMA,
        ],
    )
    w16 = jnp.pad(w, (0, 16 - w.shape[0]))
    h_i = sample[:, 0]
    r_i = sample[:, 1]
    t_i = sample[:, 2]
    return run(h_i, r_i, t_i, ent0, ent1, ent4, ent5, rel0, rel1, rel3, rel4,
               w16)
